# baseline (speedup 1.0000x reference)
import jax
import jax.numpy as jnp
from jax.experimental import pallas as pl
from jax.experimental.pallas import tpu as pltpu

N_NODES_ = 100000
N_EDGES_ = 6400000


def _mlp2_block(x_ref, agg_ref, W2a_ref, b2a_ref, W2b_ref, b2b_ref, o_ref):
    h = jnp.concatenate([x_ref[...], agg_ref[...]], axis=1)
    h = jnp.maximum(h @ W2a_ref[...] + b2a_ref[...], 0.0)
    h = jnp.maximum(h @ W2b_ref[...] + b2b_ref[...], 0.0)
    o_ref[...] = jnp.concatenate([x_ref[:, :-1], h], axis=1)


def _conv(x, src, dst, edge_attr, W1a, b1a, W1b, b1b, W2a, b2a, W2b, b2b):
    x_j = jnp.take(x, src, axis=0)
    h = jnp.concatenate([x_j, edge_attr], axis=1)
    h = jnp.maximum(h @ W1a + b1a, 0.0)
    msg = jnp.maximum(h @ W1b + b1b, 0.0)
    agg = jax.ops.segment_max(msg, dst, num_segments=N_NODES_)
    agg = jnp.where(jnp.isfinite(agg), agg, 0.0)
    blk = 8192
    npad = ((N_NODES_ + blk - 1) // blk) * blk
    xp = jnp.pad(x, ((0, npad - N_NODES_), (0, 0)))
    ap = jnp.pad(agg, ((0, npad - N_NODES_), (0, 0)))
    out = pl.pallas_call(
        _mlp2_block,
        grid=(npad // blk,),
        in_specs=[
            pl.BlockSpec((blk, 12), lambda i: (i, 0)),
            pl.BlockSpec((blk, 32), lambda i: (i, 0)),
            pl.BlockSpec((44, 16), lambda i: (0, 0)),
            pl.BlockSpec((16,), lambda i: (0,)),
            pl.BlockSpec((16, 1), lambda i: (0, 0)),
            pl.BlockSpec((1,), lambda i: (0,)),
        ],
        out_specs=pl.BlockSpec((blk, 12), lambda i: (i, 0)),
        out_shape=jax.ShapeDtypeStruct((npad, 12), jnp.float32),
    )(xp, ap, W2a, b2a, W2b, b2b)
    return out[:N_NODES_]


def kernel(x, edge_index, edge_attr, W1a, b1a, W1b, b1b, W2a, b2a, W2b, b2b):
    src = edge_index[0].astype(jnp.int32)
    dst = edge_index[1].astype(jnp.int32)
    x1 = _conv(x, src, dst, edge_attr, W1a, b1a, W1b, b1b, W2a, b2a, W2b, b2b)
    x2 = _conv(x1, src, dst, edge_attr, W1a, b1a, W1b, b1b, W2a, b2a, W2b, b2b)
    out = _conv(x2, src, dst, edge_attr, W1a, b1a, W1b, b1b, W2a, b2a, W2b, b2b)
    return out


# speedup vs baseline: 1.9131x; 1.9131x over previous
import functools

import jax
import jax.numpy as jnp
from jax import lax
from jax.experimental import pallas as pl
from jax.experimental.pallas import tpu as pltpu
from jax.experimental.pallas import tpu_sc as plsc

NN = 100000
NE = 6400000
NBLK = 2048
NPAD = 100352
EPAD = 6422528
NWORK = 32
EW = EPAD // NWORK
GCH = 1024
GSUB = 128
UCH = 4096
UPAIRS = EPAD // (2 * UCH)
TRASH = NN

_mesh = functools.partial(
    plsc.VectorSubcoreMesh,
    core_axis_name="c",
    subcore_axis_name="s",
    num_cores=2,
    num_subcores=16,
)


def _node_body(x_ref, w1x_ref, w1xz_ref, w2xz_ref, b1_ref, b2_ref,
               pre0_ref, w1b_ref, w2b_ref):
    xb = x_ref[...]
    pre0_ref[...] = xb @ w1x_ref[...] + b1_ref[...]
    w1b_ref[...] = xb @ w1xz_ref[...] + b1_ref[...]
    w2b_ref[...] = xb @ w2xz_ref[...] + b2_ref[...]


def _node_premix(xp, w1x, w1xz, w2xz, b1, b2):
    return pl.pallas_call(
        _node_body,
        grid=(NPAD // NBLK,),
        in_specs=[
            pl.BlockSpec((NBLK, 12), lambda i: (i, 0)),
            pl.BlockSpec((12, 16), lambda i: (0, 0)),
            pl.BlockSpec((12, 16), lambda i: (0, 0)),
            pl.BlockSpec((12, 16), lambda i: (0, 0)),
            pl.BlockSpec((1, 16), lambda i: (0, 0)),
            pl.BlockSpec((1, 16), lambda i: (0, 0)),
        ],
        out_specs=[
            pl.BlockSpec((NBLK, 16), lambda i: (i, 0)),
            pl.BlockSpec((NBLK, 16), lambda i: (i, 0)),
            pl.BlockSpec((NBLK, 16), lambda i: (i, 0)),
        ],
        out_shape=[
            jax.ShapeDtypeStruct((NPAD, 16), jnp.float32),
            jax.ShapeDtypeStruct((NPAD, 16), jnp.float32),
            jax.ShapeDtypeStruct((NPAD, 16), jnp.float32),
        ],
    )(xp, w1x, w1xz, w2xz, b1, b2)


def _gather_body(pre_hbm, src_hbm, out_hbm, idx_v, rows_v, sem):
    w = lax.axis_index("s") * 2 + lax.axis_index("c")
    base0 = w * EW

    def chunk(ci, _):
        base = base0 + ci * GCH
        pltpu.sync_copy(src_hbm.at[pl.ds(base, GCH)], idx_v)
        descs = []
        for j in range(GCH // GSUB):
            descs.append(pltpu.async_copy(
                pre_hbm.at[idx_v.at[pl.ds(j * GSUB, GSUB)]],
                rows_v.at[pl.ds(j * GSUB, GSUB), :],
                sem,
            ))
        for d in descs:
            d.wait()
        pltpu.sync_copy(rows_v, out_hbm.at[pl.ds(base, GCH)])
        return 0

    lax.fori_loop(0, EW // GCH, chunk, 0)


def _gather(pre, srcp):
    return pl.kernel(
        _gather_body,
        out_type=jax.ShapeDtypeStruct((EPAD, 16), jnp.float32),
        mesh=_mesh(),
        compiler_params=pltpu.CompilerParams(use_tc_tiling_on_sc=False),
        scratch_types=[
            pltpu.VMEM((GCH,), jnp.int32),
            pltpu.VMEM((GCH, 16), jnp.float32),
            pltpu.SemaphoreType.DMA,
        ],
    )(pre, srcp)


def _msg_body(g_ref, ea_ref, wea_ref, w1b_ref, b1b_ref, o_ref):
    eac = lax.dot_general(ea_ref[...], wea_ref[...], (((0,), (0,)), ((), ())))
    h1 = jnp.maximum(g_ref[...] + eac, 0.0)
    m = lax.dot_general(w1b_ref[...], h1, (((0,), (1,)), ((), ())))
    o_ref[...] = jnp.maximum(m + b1b_ref[...], 0.0)


def _msg(gathered, eaT, wea, w1b, b1b_col):
    eblk = 4096
    return pl.pallas_call(
        _msg_body,
        grid=(EPAD // eblk,),
        in_specs=[
            pl.BlockSpec((eblk, 16), lambda i: (i, 0)),
            pl.BlockSpec((2, eblk), lambda i: (0, i)),
            pl.BlockSpec((2, 16), lambda i: (0, 0)),
            pl.BlockSpec((16, 32), lambda i: (0, 0)),
            pl.BlockSpec((32, 1), lambda i: (0, 0)),
        ],
        out_specs=pl.BlockSpec((32, eblk), lambda i: (0, i)),
        out_shape=jax.ShapeDtypeStruct((32, EPAD), jnp.float32),
    )(gathered, eaT, wea, w1b, b1b_col)


def _update_body(msg_hbm, dst_hbm, out_hbm,
                 table, dst0, dst1, msg0, msg1, sem0, sem1):
    f = lax.axis_index("s") * 2 + lax.axis_index("c")

    def start(ci, dstb, msgb, sem):
        eb = ci * UCH
        pltpu.async_copy(dst_hbm.at[pl.ds(eb, UCH)], dstb, sem)
        pltpu.async_copy(msg_hbm.at[f, pl.ds(eb, UCH)], msgb, sem)

    def drain(dstb, msgb, sem):
        pltpu.make_async_copy(dst_hbm.at[pl.ds(0, UCH)], dstb, sem).wait()
        pltpu.make_async_copy(msg_hbm.at[f, pl.ds(0, UCH)], msgb, sem).wait()

    def compute(dstb, msgb):
        def vreg(g, _):
            dstv = dstb[pl.ds(g * 16, 16)]
            m = msgb[pl.ds(g * 16, 16)]
            occ, _last = plsc.scan_count(dstv)
            mx = jnp.max(occ)

            def round_body(r):
                msk = occ == r
                cur = plsc.load_gather(table, [dstv], mask=msk)
                plsc.store_scatter(table, [dstv], jnp.maximum(cur, m),
                                   mask=msk)
                return r + 1

            lax.while_loop(lambda r: r <= mx, round_body, jnp.min(occ))
            return 0

        lax.fori_loop(0, UCH // 16, vreg, 0)

    def init(i, _):
        table[pl.ds(i * 16, 16)] = jnp.full((16,), -1.0, jnp.float32)
        return 0

    lax.fori_loop(0, NPAD // 16, init, 0)

    start(0, dst0, msg0, sem0)

    def pair(i, _):
        start(2 * i + 1, dst1, msg1, sem1)
        drain(dst0, msg0, sem0)
        compute(dst0, msg0)

        @pl.when(i < UPAIRS - 1)
        def _():
            start(2 * i + 2, dst0, msg0, sem0)

        drain(dst1, msg1, sem1)
        compute(dst1, msg1)
        return 0

    lax.fori_loop(0, UPAIRS, pair, 0)
    pltpu.sync_copy(table, out_hbm.at[f])


def _update(msgsT, dstp):
    return pl.kernel(
        _update_body,
        out_type=jax.ShapeDtypeStruct((32, NPAD), jnp.float32),
        mesh=_mesh(),
        compiler_params=pltpu.CompilerParams(needs_layout_passes=False),
        scratch_types=[
            pltpu.VMEM((NPAD,), jnp.float32),
            pltpu.VMEM((UCH,), jnp.int32),
            pltpu.VMEM((UCH,), jnp.int32),
            pltpu.VMEM((UCH,), jnp.float32),
            pltpu.VMEM((UCH,), jnp.float32),
            pltpu.SemaphoreType.DMA,
            pltpu.SemaphoreType.DMA,
        ],
    )(msgsT, dstp)


def _mlp2_body(aggT_ref, w2b_ref, w1b_ref, x11_ref, w2agg_ref, w2o_ref,
               b2o_ref, w1a11_ref, w2a11_ref, pre_ref, comb_ref):
    agg = jnp.maximum(aggT_ref[...], 0.0)
    t = jnp.maximum(
        w2b_ref[...] + x11_ref[...] * w2a11_ref[...]
        + lax.dot_general(agg, w2agg_ref[...], (((0,), (0,)), ((), ()))),
        0.0)
    comb = jnp.maximum(t @ w2o_ref[...] + b2o_ref[...], 0.0)
    pre_ref[...] = w1b_ref[...] + comb * w1a11_ref[...]
    comb_ref[...] = comb


def _mlp2(aggT, w2base, w1base, x11, w2agg, w2o, b2o, w1a11, w2a11):
    return pl.pallas_call(
        _mlp2_body,
        grid=(NPAD // NBLK,),
        in_specs=[
            pl.BlockSpec((32, NBLK), lambda i: (0, i)),
            pl.BlockSpec((NBLK, 16), lambda i: (i, 0)),
            pl.BlockSpec((NBLK, 16), lambda i: (i, 0)),
            pl.BlockSpec((NBLK, 1), lambda i: (i, 0)),
            pl.BlockSpec((32, 16), lambda i: (0, 0)),
            pl.BlockSpec((16, 1), lambda i: (0, 0)),
            pl.BlockSpec((1, 1), lambda i: (0, 0)),
            pl.BlockSpec((1, 16), lambda i: (0, 0)),
            pl.BlockSpec((1, 16), lambda i: (0, 0)),
        ],
        out_specs=[
            pl.BlockSpec((NBLK, 16), lambda i: (i, 0)),
            pl.BlockSpec((NBLK, 1), lambda i: (i, 0)),
        ],
        out_shape=[
            jax.ShapeDtypeStruct((NPAD, 16), jnp.float32),
            jax.ShapeDtypeStruct((NPAD, 1), jnp.float32),
        ],
    )(aggT, w2base, w1base, x11, w2agg, w2o, b2o, w1a11, w2a11)


def _final_body(aggT_ref, w2b_ref, x_ref, x11_ref, w2agg_ref, w2o_ref,
                b2o_ref, w2a11_ref, o_ref):
    agg = jnp.maximum(aggT_ref[...], 0.0)
    t = jnp.maximum(
        w2b_ref[...] + x11_ref[...] * w2a11_ref[...]
        + lax.dot_general(agg, w2agg_ref[...], (((0,), (0,)), ((), ()))),
        0.0)
    comb = jnp.maximum(t @ w2o_ref[...] + b2o_ref[...], 0.0)
    o_ref[...] = jnp.concatenate([x_ref[...][:, :11], comb], axis=1)


def _final(aggT, w2base, xp, x11, w2agg, w2o, b2o, w2a11):
    return pl.pallas_call(
        _final_body,
        grid=(NPAD // NBLK,),
        in_specs=[
            pl.BlockSpec((32, NBLK), lambda i: (0, i)),
            pl.BlockSpec((NBLK, 16), lambda i: (i, 0)),
            pl.BlockSpec((NBLK, 12), lambda i: (i, 0)),
            pl.BlockSpec((NBLK, 1), lambda i: (i, 0)),
            pl.BlockSpec((32, 16), lambda i: (0, 0)),
            pl.BlockSpec((16, 1), lambda i: (0, 0)),
            pl.BlockSpec((1, 1), lambda i: (0, 0)),
            pl.BlockSpec((1, 16), lambda i: (0, 0)),
        ],
        out_specs=pl.BlockSpec((NBLK, 12), lambda i: (i, 0)),
        out_shape=jax.ShapeDtypeStruct((NPAD, 12), jnp.float32),
    )(aggT, w2base, xp, x11, w2agg, w2o, b2o, w2a11)


def kernel(x, edge_index, edge_attr, W1a, b1a, W1b, b1b, W2a, b2a, W2b, b2b):
    src = edge_index[0].astype(jnp.int32)
    dst = edge_index[1].astype(jnp.int32)

    xp = jnp.pad(x, ((0, NPAD - NN), (0, 0)))
    srcp = jnp.pad(src, (0, EPAD - NE))
    dstp = jnp.pad(dst, (0, EPAD - NE), constant_values=TRASH)
    eaT = jnp.pad(edge_attr.T, ((0, 0), (0, EPAD - NE)))

    zrow = jnp.zeros((1, 16), jnp.float32)
    w1x = W1a[:12]
    w1xz = jnp.concatenate([W1a[:11], zrow], axis=0)
    wea = W1a[12:14]
    w2xz = jnp.concatenate([W2a[:11], zrow], axis=0)
    w2agg = W2a[12:44]
    b1 = b1a[None, :]
    b2 = b2a[None, :]
    b1b_col = b1b[:, None]
    b2o = b2b[None, :]
    w1a11 = W1a[11:12]
    w2a11 = W2a[11:12]

    pre, w1base, w2base = _node_premix(xp, w1x, w1xz, w2xz, b1, b2)
    x11 = xp[:, 11:12]

    for layer in range(3):
        gathered = _gather(pre, srcp)
        msgsT = _msg(gathered, eaT, wea, W1b, b1b_col)
        aggT = _update(msgsT, dstp)
        if layer < 2:
            pre, x11 = _mlp2(aggT, w2base, w1base, x11, w2agg, W2b, b2o,
                             w1a11, w2a11)
        else:
            outp = _final(aggT, w2base, xp, x11, w2agg, W2b, b2o, w2a11)

    return outp[:NN]


# speedup vs baseline: 2.7308x; 1.4274x over previous
import functools

import jax
import jax.numpy as jnp
from jax import lax
from jax.experimental import pallas as pl
from jax.experimental.pallas import tpu as pltpu
from jax.experimental.pallas import tpu_sc as plsc

NN = 100000
NE = 6400000
NBLK = 2048
NPAD = 100352
EPAD = 6422528
NWORK = 32
EW = EPAD // NWORK
GCH = 1024
GSUB = 128
UCH = 4096
UPAIRS = EPAD // (2 * UCH)
TRASH = NN

_mesh = functools.partial(
    plsc.VectorSubcoreMesh,
    core_axis_name="c",
    subcore_axis_name="s",
    num_cores=2,
    num_subcores=16,
)


def _node_body(x_ref, w1x_ref, w1xz_ref, w2xz_ref, b1_ref, b2_ref,
               pre0_ref, w1b_ref, w2b_ref):
    xb = x_ref[...]
    pre0_ref[...] = xb @ w1x_ref[...] + b1_ref[...]
    w1b_ref[...] = xb @ w1xz_ref[...] + b1_ref[...]
    w2b_ref[...] = xb @ w2xz_ref[...] + b2_ref[...]


def _node_premix(xp, w1x, w1xz, w2xz, b1, b2):
    return pl.pallas_call(
        _node_body,
        grid=(NPAD // NBLK,),
        in_specs=[
            pl.BlockSpec((NBLK, 12), lambda i: (i, 0)),
            pl.BlockSpec((12, 16), lambda i: (0, 0)),
            pl.BlockSpec((12, 16), lambda i: (0, 0)),
            pl.BlockSpec((12, 16), lambda i: (0, 0)),
            pl.BlockSpec((1, 16), lambda i: (0, 0)),
            pl.BlockSpec((1, 16), lambda i: (0, 0)),
        ],
        out_specs=[
            pl.BlockSpec((NBLK, 16), lambda i: (i, 0)),
            pl.BlockSpec((NBLK, 16), lambda i: (i, 0)),
            pl.BlockSpec((NBLK, 16), lambda i: (i, 0)),
        ],
        out_shape=[
            jax.ShapeDtypeStruct((NPAD, 16), jnp.float32),
            jax.ShapeDtypeStruct((NPAD, 16), jnp.float32),
            jax.ShapeDtypeStruct((NPAD, 16), jnp.float32),
        ],
    )(xp, w1x, w1xz, w2xz, b1, b2)


def _gather_body(pre_hbm, src_hbm, out_hbm, idx_v, rows_v, sem):
    w = lax.axis_index("s") * 2 + lax.axis_index("c")
    base0 = w * EW

    def chunk(ci, _):
        base = base0 + ci * GCH
        pltpu.sync_copy(src_hbm.at[pl.ds(base, GCH)], idx_v)
        descs = []
        for j in range(GCH // GSUB):
            descs.append(pltpu.async_copy(
                pre_hbm.at[idx_v.at[pl.ds(j * GSUB, GSUB)]],
                rows_v.at[pl.ds(j * GSUB, GSUB), :],
                sem,
            ))
        for d in descs:
            d.wait()
        pltpu.sync_copy(rows_v, out_hbm.at[pl.ds(base, GCH)])
        return 0

    lax.fori_loop(0, EW // GCH, chunk, 0)


def _gather(pre, srcp):
    return pl.kernel(
        _gather_body,
        out_type=jax.ShapeDtypeStruct((EPAD, 16), jnp.float32),
        mesh=_mesh(),
        compiler_params=pltpu.CompilerParams(use_tc_tiling_on_sc=False),
        scratch_types=[
            pltpu.VMEM((GCH,), jnp.int32),
            pltpu.VMEM((GCH, 16), jnp.float32),
            pltpu.SemaphoreType.DMA,
        ],
    )(pre, srcp)


def _msg_body(g_ref, ea_ref, wea_ref, w1b_ref, b1b_ref, o_ref):
    eac = lax.dot_general(ea_ref[...], wea_ref[...], (((0,), (0,)), ((), ())))
    h1 = jnp.maximum(g_ref[...] + eac, 0.0)
    m = lax.dot_general(w1b_ref[...], h1, (((0,), (1,)), ((), ())))
    o_ref[...] = jnp.maximum(m + b1b_ref[...], 0.0)


def _msg(gathered, eaT, wea, w1b, b1b_col):
    eblk = 4096
    return pl.pallas_call(
        _msg_body,
        grid=(EPAD // eblk,),
        in_specs=[
            pl.BlockSpec((eblk, 16), lambda i: (i, 0)),
            pl.BlockSpec((2, eblk), lambda i: (0, i)),
            pl.BlockSpec((2, 16), lambda i: (0, 0)),
            pl.BlockSpec((16, 32), lambda i: (0, 0)),
            pl.BlockSpec((32, 1), lambda i: (0, 0)),
        ],
        out_specs=pl.BlockSpec((32, eblk), lambda i: (0, i)),
        out_shape=jax.ShapeDtypeStruct((32, EPAD), jnp.float32),
    )(gathered, eaT, wea, w1b, b1b_col)


def _update_body(msg_hbm, dst_hbm, out_hbm,
                 table, dst0, dst1, msg0, msg1, sem0, sem1):
    f = lax.axis_index("s") * 2 + lax.axis_index("c")

    def start(ci, dstb, msgb, sem):
        eb = ci * UCH
        pltpu.async_copy(dst_hbm.at[pl.ds(eb, UCH)], dstb, sem)
        pltpu.async_copy(msg_hbm.at[f, pl.ds(eb, UCH)], msgb, sem)

    def drain(dstb, msgb, sem):
        pltpu.make_async_copy(dst_hbm.at[pl.ds(0, UCH)], dstb, sem).wait()
        pltpu.make_async_copy(msg_hbm.at[f, pl.ds(0, UCH)], msgb, sem).wait()

    def compute(dstb, msgb):
        def vreg(g, acc):
            dstv = dstb[pl.ds(g * 16, 16)]
            m = msgb[pl.ds(g * 16, 16)]
            occ, _last = plsc.scan_count(dstv)
            for r in (1, 2):
                msk = occ == r
                cur = plsc.load_gather(table, [dstv], mask=msk)
                plsc.store_scatter(table, [dstv], jnp.maximum(cur, m),
                                   mask=msk)
            return acc | (occ > 2).astype(jnp.int32)

        acc = lax.fori_loop(0, UCH // 16, vreg,
                            jnp.zeros((16,), jnp.int32))

        @pl.when(jnp.max(acc) > 0)
        def _():
            def vreg_slow(g, _):
                dstv = dstb[pl.ds(g * 16, 16)]
                m = msgb[pl.ds(g * 16, 16)]
                occ, _last = plsc.scan_count(dstv)
                mx = jnp.max(occ)

                def round_body(r):
                    msk = occ == r
                    cur = plsc.load_gather(table, [dstv], mask=msk)
                    plsc.store_scatter(table, [dstv], jnp.maximum(cur, m),
                                       mask=msk)
                    return r + 1

                lax.while_loop(lambda r: r <= mx, round_body,
                               jnp.int32(3))
                return 0

            lax.fori_loop(0, UCH // 16, vreg_slow, 0)

    def init(i, _):
        table[pl.ds(i * 16, 16)] = jnp.full((16,), -1.0, jnp.float32)
        return 0

    lax.fori_loop(0, NPAD // 16, init, 0)

    start(0, dst0, msg0, sem0)

    def pair(i, _):
        start(2 * i + 1, dst1, msg1, sem1)
        drain(dst0, msg0, sem0)
        compute(dst0, msg0)

        @pl.when(i < UPAIRS - 1)
        def _():
            start(2 * i + 2, dst0, msg0, sem0)

        drain(dst1, msg1, sem1)
        compute(dst1, msg1)
        return 0

    lax.fori_loop(0, UPAIRS, pair, 0)
    pltpu.sync_copy(table, out_hbm.at[f])


def _update(msgsT, dstp):
    return pl.kernel(
        _update_body,
        out_type=jax.ShapeDtypeStruct((32, NPAD), jnp.float32),
        mesh=_mesh(),
        compiler_params=pltpu.CompilerParams(needs_layout_passes=False),
        scratch_types=[
            pltpu.VMEM((NPAD,), jnp.float32),
            pltpu.VMEM((UCH,), jnp.int32),
            pltpu.VMEM((UCH,), jnp.int32),
            pltpu.VMEM((UCH,), jnp.float32),
            pltpu.VMEM((UCH,), jnp.float32),
            pltpu.SemaphoreType.DMA,
            pltpu.SemaphoreType.DMA,
        ],
    )(msgsT, dstp)


def _mlp2_body(aggT_ref, w2b_ref, w1b_ref, x11_ref, w2agg_ref, w2o_ref,
               b2o_ref, w1a11_ref, w2a11_ref, pre_ref, comb_ref):
    agg = jnp.maximum(aggT_ref[...], 0.0)
    t = jnp.maximum(
        w2b_ref[...] + x11_ref[...] * w2a11_ref[...]
        + lax.dot_general(agg, w2agg_ref[...], (((0,), (0,)), ((), ()))),
        0.0)
    comb = jnp.maximum(t @ w2o_ref[...] + b2o_ref[...], 0.0)
    pre_ref[...] = w1b_ref[...] + comb * w1a11_ref[...]
    comb_ref[...] = comb


def _mlp2(aggT, w2base, w1base, x11, w2agg, w2o, b2o, w1a11, w2a11):
    return pl.pallas_call(
        _mlp2_body,
        grid=(NPAD // NBLK,),
        in_specs=[
            pl.BlockSpec((32, NBLK), lambda i: (0, i)),
            pl.BlockSpec((NBLK, 16), lambda i: (i, 0)),
            pl.BlockSpec((NBLK, 16), lambda i: (i, 0)),
            pl.BlockSpec((NBLK, 1), lambda i: (i, 0)),
            pl.BlockSpec((32, 16), lambda i: (0, 0)),
            pl.BlockSpec((16, 1), lambda i: (0, 0)),
            pl.BlockSpec((1, 1), lambda i: (0, 0)),
            pl.BlockSpec((1, 16), lambda i: (0, 0)),
            pl.BlockSpec((1, 16), lambda i: (0, 0)),
        ],
        out_specs=[
            pl.BlockSpec((NBLK, 16), lambda i: (i, 0)),
            pl.BlockSpec((NBLK, 1), lambda i: (i, 0)),
        ],
        out_shape=[
            jax.ShapeDtypeStruct((NPAD, 16), jnp.float32),
            jax.ShapeDtypeStruct((NPAD, 1), jnp.float32),
        ],
    )(aggT, w2base, w1base, x11, w2agg, w2o, b2o, w1a11, w2a11)


def _final_body(aggT_ref, w2b_ref, x_ref, x11_ref, w2agg_ref, w2o_ref,
                b2o_ref, w2a11_ref, o_ref):
    agg = jnp.maximum(aggT_ref[...], 0.0)
    t = jnp.maximum(
        w2b_ref[...] + x11_ref[...] * w2a11_ref[...]
        + lax.dot_general(agg, w2agg_ref[...], (((0,), (0,)), ((), ()))),
        0.0)
    comb = jnp.maximum(t @ w2o_ref[...] + b2o_ref[...], 0.0)
    o_ref[...] = jnp.concatenate([x_ref[...][:, :11], comb], axis=1)


def _final(aggT, w2base, xp, x11, w2agg, w2o, b2o, w2a11):
    return pl.pallas_call(
        _final_body,
        grid=(NPAD // NBLK,),
        in_specs=[
            pl.BlockSpec((32, NBLK), lambda i: (0, i)),
            pl.BlockSpec((NBLK, 16), lambda i: (i, 0)),
            pl.BlockSpec((NBLK, 12), lambda i: (i, 0)),
            pl.BlockSpec((NBLK, 1), lambda i: (i, 0)),
            pl.BlockSpec((32, 16), lambda i: (0, 0)),
            pl.BlockSpec((16, 1), lambda i: (0, 0)),
            pl.BlockSpec((1, 1), lambda i: (0, 0)),
            pl.BlockSpec((1, 16), lambda i: (0, 0)),
        ],
        out_specs=pl.BlockSpec((NBLK, 12), lambda i: (i, 0)),
        out_shape=jax.ShapeDtypeStruct((NPAD, 12), jnp.float32),
    )(aggT, w2base, xp, x11, w2agg, w2o, b2o, w2a11)


def kernel(x, edge_index, edge_attr, W1a, b1a, W1b, b1b, W2a, b2a, W2b, b2b):
    src = edge_index[0].astype(jnp.int32)
    dst = edge_index[1].astype(jnp.int32)

    xp = jnp.pad(x, ((0, NPAD - NN), (0, 0)))
    srcp = jnp.pad(src, (0, EPAD - NE))
    dstp = jnp.pad(dst, (0, EPAD - NE), constant_values=TRASH)
    eaT = jnp.pad(edge_attr.T, ((0, 0), (0, EPAD - NE)))

    zrow = jnp.zeros((1, 16), jnp.float32)
    w1x = W1a[:12]
    w1xz = jnp.concatenate([W1a[:11], zrow], axis=0)
    wea = W1a[12:14]
    w2xz = jnp.concatenate([W2a[:11], zrow], axis=0)
    w2agg = W2a[12:44]
    b1 = b1a[None, :]
    b2 = b2a[None, :]
    b1b_col = b1b[:, None]
    b2o = b2b[None, :]
    w1a11 = W1a[11:12]
    w2a11 = W2a[11:12]

    pre, w1base, w2base = _node_premix(xp, w1x, w1xz, w2xz, b1, b2)
    x11 = xp[:, 11:12]

    for layer in range(3):
        gathered = _gather(pre, srcp)
        msgsT = _msg(gathered, eaT, wea, W1b, b1b_col)
        aggT = _update(msgsT, dstp)
        if layer < 2:
            pre, x11 = _mlp2(aggT, w2base, w1base, x11, w2agg, W2b, b2o,
                             w1a11, w2a11)
        else:
            outp = _final(aggT, w2base, xp, x11, w2agg, W2b, b2o, w2a11)

    return outp[:NN]


# speedup vs baseline: 3.8773x; 1.4198x over previous
import functools

import jax
import jax.numpy as jnp
from jax import lax
from jax.experimental import pallas as pl
from jax.experimental.pallas import tpu as pltpu
from jax.experimental.pallas import tpu_sc as plsc

NN = 100000
NE = 6400000
NBLK = 2048
NPAD = 100352
EPAD = 6422528
NWORK = 32
EW = EPAD // NWORK
GCH = 1024
GSUB = 128
UCH = 4096
UPAIRS_H = EPAD // (4 * UCH)
TRASH = NN

_mesh = functools.partial(
    plsc.VectorSubcoreMesh,
    core_axis_name="c",
    subcore_axis_name="s",
    num_cores=2,
    num_subcores=16,
)


def _node_body(x_ref, w1x_ref, w1xz_ref, w2xz_ref, b1_ref, b2_ref,
               pre0_ref, w1b_ref, w2b_ref):
    xb = x_ref[...]
    pre0_ref[...] = xb @ w1x_ref[...] + b1_ref[...]
    w1b_ref[...] = xb @ w1xz_ref[...] + b1_ref[...]
    w2b_ref[...] = xb @ w2xz_ref[...] + b2_ref[...]


def _node_premix(xp, w1x, w1xz, w2xz, b1, b2):
    return pl.pallas_call(
        _node_body,
        grid=(NPAD // NBLK,),
        in_specs=[
            pl.BlockSpec((NBLK, 12), lambda i: (i, 0)),
            pl.BlockSpec((12, 16), lambda i: (0, 0)),
            pl.BlockSpec((12, 16), lambda i: (0, 0)),
            pl.BlockSpec((12, 16), lambda i: (0, 0)),
            pl.BlockSpec((1, 16), lambda i: (0, 0)),
            pl.BlockSpec((1, 16), lambda i: (0, 0)),
        ],
        out_specs=[
            pl.BlockSpec((NBLK, 16), lambda i: (i, 0)),
            pl.BlockSpec((NBLK, 16), lambda i: (i, 0)),
            pl.BlockSpec((NBLK, 16), lambda i: (i, 0)),
        ],
        out_shape=[
            jax.ShapeDtypeStruct((NPAD, 16), jnp.float32),
            jax.ShapeDtypeStruct((NPAD, 16), jnp.float32),
            jax.ShapeDtypeStruct((NPAD, 16), jnp.float32),
        ],
    )(xp, w1x, w1xz, w2xz, b1, b2)


def _gather_body(pre_hbm, src_hbm, out_hbm, idx_v, rows_v, sem):
    w = lax.axis_index("s") * 2 + lax.axis_index("c")
    base0 = w * EW

    def chunk(ci, _):
        base = base0 + ci * GCH
        pltpu.sync_copy(src_hbm.at[pl.ds(base, GCH)], idx_v)
        descs = []
        for j in range(GCH // GSUB):
            descs.append(pltpu.async_copy(
                pre_hbm.at[idx_v.at[pl.ds(j * GSUB, GSUB)]],
                rows_v.at[pl.ds(j * GSUB, GSUB), :],
                sem,
            ))
        for d in descs:
            d.wait()
        pltpu.sync_copy(rows_v, out_hbm.at[pl.ds(base, GCH)])
        return 0

    lax.fori_loop(0, EW // GCH, chunk, 0)


def _gather(pre, srcp):
    return pl.kernel(
        _gather_body,
        out_type=jax.ShapeDtypeStruct((EPAD, 16), jnp.float32),
        mesh=_mesh(),
        compiler_params=pltpu.CompilerParams(use_tc_tiling_on_sc=False),
        scratch_types=[
            pltpu.VMEM((GCH,), jnp.int32),
            pltpu.VMEM((GCH, 16), jnp.float32),
            pltpu.SemaphoreType.DMA,
        ],
    )(pre, srcp)


def _msg_body(g_ref, ea_ref, wea_ref, w1be_ref, w1bo_ref, b1be_ref,
              b1bo_ref, o_ref):
    eac = lax.dot_general(ea_ref[...], wea_ref[...], (((0,), (0,)), ((), ())))
    h1 = jnp.maximum(g_ref[...] + eac, 0.0)
    me = jnp.maximum(
        lax.dot_general(w1be_ref[...], h1, (((0,), (1,)), ((), ())))
        + b1be_ref[...], 0.0)
    mo = jnp.maximum(
        lax.dot_general(w1bo_ref[...], h1, (((0,), (1,)), ((), ())))
        + b1bo_ref[...], 0.0)
    ue = lax.shift_right_logical(
        lax.bitcast_convert_type(me.astype(jnp.bfloat16).astype(jnp.float32),
                                 jnp.int32), 16)
    uo = lax.shift_right_logical(
        lax.bitcast_convert_type(mo.astype(jnp.bfloat16).astype(jnp.float32),
                                 jnp.int32), 16)
    o_ref[...] = ue | lax.shift_left(uo, 16)


def _msg(gathered, eaT, wea, w1be, w1bo, b1be, b1bo):
    eblk = 4096
    return pl.pallas_call(
        _msg_body,
        grid=(EPAD // eblk,),
        in_specs=[
            pl.BlockSpec((eblk, 16), lambda i: (i, 0)),
            pl.BlockSpec((2, eblk), lambda i: (0, i)),
            pl.BlockSpec((2, 16), lambda i: (0, 0)),
            pl.BlockSpec((16, 16), lambda i: (0, 0)),
            pl.BlockSpec((16, 16), lambda i: (0, 0)),
            pl.BlockSpec((16, 1), lambda i: (0, 0)),
            pl.BlockSpec((16, 1), lambda i: (0, 0)),
        ],
        out_specs=pl.BlockSpec((16, eblk), lambda i: (0, i)),
        out_shape=jax.ShapeDtypeStruct((16, EPAD), jnp.int32),
    )(gathered, eaT, wea, w1be, w1bo, b1be, b1bo)


def _pmax(a, b):
    lo = jnp.maximum(a & 0xFFFF, b & 0xFFFF)
    hi = jnp.maximum(lax.shift_right_logical(a, 16),
                     lax.shift_right_logical(b, 16))
    return lo | lax.shift_left(hi, 16)


def _update_body(msg_hbm, dst_hbm, out_hbm,
                 table, dst0, dst1, msg0, msg1, sem0, sem1):
    s = lax.axis_index("s")
    c = lax.axis_index("c")
    cbase = c * (EPAD // UCH // 2)

    def start(ci, dstb, msgb, sem):
        eb = ci * UCH
        pltpu.async_copy(dst_hbm.at[pl.ds(eb, UCH)], dstb, sem)
        pltpu.async_copy(msg_hbm.at[s, pl.ds(eb, UCH)], msgb, sem)

    def drain(dstb, msgb, sem):
        pltpu.make_async_copy(dst_hbm.at[pl.ds(0, UCH)], dstb, sem).wait()
        pltpu.make_async_copy(msg_hbm.at[s, pl.ds(0, UCH)], msgb, sem).wait()

    def compute(dstb, msgb):
        def vreg(g, acc):
            dstv = dstb[pl.ds(g * 16, 16)]
            m = msgb[pl.ds(g * 16, 16)]
            occ, _last = plsc.scan_count(dstv)
            for r in (1, 2):
                msk = occ == r
                cur = plsc.load_gather(table, [dstv], mask=msk)
                plsc.store_scatter(table, [dstv], _pmax(cur, m), mask=msk)
            return acc | (occ > 2).astype(jnp.int32)

        acc = lax.fori_loop(0, UCH // 16, vreg,
                            jnp.zeros((16,), jnp.int32))

        @pl.when(jnp.max(acc) > 0)
        def _():
            def vreg_slow(g, _):
                dstv = dstb[pl.ds(g * 16, 16)]
                m = msgb[pl.ds(g * 16, 16)]
                occ, _last = plsc.scan_count(dstv)
                mx = jnp.max(occ)

                def round_body(r):
                    msk = occ == r
                    cur = plsc.load_gather(table, [dstv], mask=msk)
                    plsc.store_scatter(table, [dstv], _pmax(cur, m),
                                       mask=msk)
                    return r + 1

                lax.while_loop(lambda r: r <= mx, round_body,
                               jnp.int32(3))
                return 0

            lax.fori_loop(0, UCH // 16, vreg_slow, 0)

    def init(i, _):
        table[pl.ds(i * 16, 16)] = jnp.zeros((16,), jnp.int32)
        return 0

    lax.fori_loop(0, NPAD // 16, init, 0)

    start(cbase, dst0, msg0, sem0)

    def pair(i, _):
        a = cbase + 2 * i
        start(a + 1, dst1, msg1, sem1)
        drain(dst0, msg0, sem0)
        compute(dst0, msg0)

        @pl.when(i < UPAIRS_H - 1)
        def _():
            start(a + 2, dst0, msg0, sem0)

        drain(dst1, msg1, sem1)
        compute(dst1, msg1)
        return 0

    lax.fori_loop(0, UPAIRS_H, pair, 0)
    pltpu.sync_copy(table, out_hbm.at[c, s])


def _update(msgsP, dstp):
    return pl.kernel(
        _update_body,
        out_type=jax.ShapeDtypeStruct((2, 16, NPAD), jnp.int32),
        mesh=_mesh(),
        compiler_params=pltpu.CompilerParams(needs_layout_passes=False),
        scratch_types=[
            pltpu.VMEM((NPAD,), jnp.int32),
            pltpu.VMEM((UCH,), jnp.int32),
            pltpu.VMEM((UCH,), jnp.int32),
            pltpu.VMEM((UCH,), jnp.int32),
            pltpu.VMEM((UCH,), jnp.int32),
            pltpu.SemaphoreType.DMA,
            pltpu.SemaphoreType.DMA,
        ],
    )(msgsP, dstp)


def _agg_terms(aggP_ref, w2e_ref, w2og_ref):
    a0 = aggP_ref[0]
    a1 = aggP_ref[1]
    ev = jnp.maximum(
        jnp.maximum(lax.bitcast_convert_type(lax.shift_left(a0, 16),
                                             jnp.float32),
                    lax.bitcast_convert_type(lax.shift_left(a1, 16),
                                             jnp.float32)), 0.0)
    od = jnp.maximum(
        jnp.maximum(
            lax.bitcast_convert_type(a0 & -65536, jnp.float32),
            lax.bitcast_convert_type(a1 & -65536, jnp.float32)), 0.0)
    return (lax.dot_general(ev, w2e_ref[...], (((0,), (0,)), ((), ())))
            + lax.dot_general(od, w2og_ref[...], (((0,), (0,)), ((), ()))))


def _mlp2_body(aggP_ref, w2b_ref, w1b_ref, x11_ref, w2e_ref, w2og_ref,
               w2o_ref, b2o_ref, w1a11_ref, w2a11_ref, pre_ref, comb_ref):
    t = jnp.maximum(
        w2b_ref[...] + x11_ref[...] * w2a11_ref[...]
        + _agg_terms(aggP_ref, w2e_ref, w2og_ref), 0.0)
    comb = jnp.maximum(t @ w2o_ref[...] + b2o_ref[...], 0.0)
    pre_ref[...] = w1b_ref[...] + comb * w1a11_ref[...]
    comb_ref[...] = comb


def _mlp2(aggP, w2base, w1base, x11, w2e, w2og, w2o, b2o, w1a11, w2a11):
    return pl.pallas_call(
        _mlp2_body,
        grid=(NPAD // NBLK,),
        in_specs=[
            pl.BlockSpec((2, 16, NBLK), lambda i: (0, 0, i)),
            pl.BlockSpec((NBLK, 16), lambda i: (i, 0)),
            pl.BlockSpec((NBLK, 16), lambda i: (i, 0)),
            pl.BlockSpec((NBLK, 1), lambda i: (i, 0)),
            pl.BlockSpec((16, 16), lambda i: (0, 0)),
            pl.BlockSpec((16, 16), lambda i: (0, 0)),
            pl.BlockSpec((16, 1), lambda i: (0, 0)),
            pl.BlockSpec((1, 1), lambda i: (0, 0)),
            pl.BlockSpec((1, 16), lambda i: (0, 0)),
            pl.BlockSpec((1, 16), lambda i: (0, 0)),
        ],
        out_specs=[
            pl.BlockSpec((NBLK, 16), lambda i: (i, 0)),
            pl.BlockSpec((NBLK, 1), lambda i: (i, 0)),
        ],
        out_shape=[
            jax.ShapeDtypeStruct((NPAD, 16), jnp.float32),
            jax.ShapeDtypeStruct((NPAD, 1), jnp.float32),
        ],
    )(aggP, w2base, w1base, x11, w2e, w2og, w2o, b2o, w1a11, w2a11)


def _final_body(aggP_ref, w2b_ref, x_ref, x11_ref, w2e_ref, w2og_ref,
                w2o_ref, b2o_ref, w2a11_ref, o_ref):
    t = jnp.maximum(
        w2b_ref[...] + x11_ref[...] * w2a11_ref[...]
        + _agg_terms(aggP_ref, w2e_ref, w2og_ref), 0.0)
    comb = jnp.maximum(t @ w2o_ref[...] + b2o_ref[...], 0.0)
    o_ref[...] = jnp.concatenate([x_ref[...][:, :11], comb], axis=1)


def _final(aggP, w2base, xp, x11, w2e, w2og, w2o, b2o, w2a11):
    return pl.pallas_call(
        _final_body,
        grid=(NPAD // NBLK,),
        in_specs=[
            pl.BlockSpec((2, 16, NBLK), lambda i: (0, 0, i)),
            pl.BlockSpec((NBLK, 16), lambda i: (i, 0)),
            pl.BlockSpec((NBLK, 12), lambda i: (i, 0)),
            pl.BlockSpec((NBLK, 1), lambda i: (i, 0)),
            pl.BlockSpec((16, 16), lambda i: (0, 0)),
            pl.BlockSpec((16, 16), lambda i: (0, 0)),
            pl.BlockSpec((16, 1), lambda i: (0, 0)),
            pl.BlockSpec((1, 1), lambda i: (0, 0)),
            pl.BlockSpec((1, 16), lambda i: (0, 0)),
        ],
        out_specs=pl.BlockSpec((NBLK, 12), lambda i: (i, 0)),
        out_shape=jax.ShapeDtypeStruct((NPAD, 12), jnp.float32),
    )(aggP, w2base, xp, x11, w2e, w2og, w2o, b2o, w2a11)


def kernel(x, edge_index, edge_attr, W1a, b1a, W1b, b1b, W2a, b2a, W2b, b2b):
    src = edge_index[0].astype(jnp.int32)
    dst = edge_index[1].astype(jnp.int32)

    xp = jnp.pad(x, ((0, NPAD - NN), (0, 0)))
    srcp = jnp.pad(src, (0, EPAD - NE))
    dstp = jnp.pad(dst, (0, EPAD - NE), constant_values=TRASH)
    eaT = jnp.pad(edge_attr.T, ((0, 0), (0, EPAD - NE)))

    zrow = jnp.zeros((1, 16), jnp.float32)
    w1x = W1a[:12]
    w1xz = jnp.concatenate([W1a[:11], zrow], axis=0)
    wea = W1a[12:14]
    w2xz = jnp.concatenate([W2a[:11], zrow], axis=0)
    w2agg = W2a[12:44]
    b1 = b1a[None, :]
    b2 = b2a[None, :]
    b2o = b2b[None, :]
    w1a11 = W1a[11:12]
    w2a11 = W2a[11:12]
    w1be = W1b[:, 0::2]
    w1bo = W1b[:, 1::2]
    b1be = b1b[0::2][:, None]
    b1bo = b1b[1::2][:, None]
    w2e = w2agg[0::2]
    w2og = w2agg[1::2]

    pre, w1base, w2base = _node_premix(xp, w1x, w1xz, w2xz, b1, b2)
    x11 = xp[:, 11:12]

    for layer in range(3):
        gathered = _gather(pre, srcp)
        msgsP = _msg(gathered, eaT, wea, w1be, w1bo, b1be, b1bo)
        aggP = _update(msgsP, dstp)
        if layer < 2:
            pre, x11 = _mlp2(aggP, w2base, w1base, x11, w2e, w2og, W2b,
                             b2o, w1a11, w2a11)
        else:
            outp = _final(aggP, w2base, xp, x11, w2e, w2og, W2b, b2o,
                          w2a11)

    return outp[:NN]


# speedup vs baseline: 5.2044x; 1.3423x over previous
import functools

import jax
import jax.numpy as jnp
from jax import lax
from jax.experimental import pallas as pl
from jax.experimental.pallas import tpu as pltpu
from jax.experimental.pallas import tpu_sc as plsc

NN = 100000
NE = 6400000
NBLK = 2048
NPAD = 100352
EPAD = 6422528
NWORK = 32
EW = EPAD // NWORK
GCH = 1024
GSUB = 128
UCH = 4096
UPAIRS_H = EPAD // (4 * UCH)
TRASH = NN

_mesh = functools.partial(
    plsc.VectorSubcoreMesh,
    core_axis_name="c",
    subcore_axis_name="s",
    num_cores=2,
    num_subcores=16,
)


def _node_body(x_ref, w1x_ref, w1xz_ref, w2xz_ref, b1_ref, b2_ref,
               pre0_ref, w1b_ref, w2b_ref):
    xb = x_ref[...]
    pre0_ref[...] = xb @ w1x_ref[...] + b1_ref[...]
    w1b_ref[...] = xb @ w1xz_ref[...] + b1_ref[...]
    w2b_ref[...] = xb @ w2xz_ref[...] + b2_ref[...]


def _node_premix(xp, w1x, w1xz, w2xz, b1, b2):
    return pl.pallas_call(
        _node_body,
        grid=(NPAD // NBLK,),
        in_specs=[
            pl.BlockSpec((NBLK, 12), lambda i: (i, 0)),
            pl.BlockSpec((12, 16), lambda i: (0, 0)),
            pl.BlockSpec((12, 16), lambda i: (0, 0)),
            pl.BlockSpec((12, 16), lambda i: (0, 0)),
            pl.BlockSpec((1, 16), lambda i: (0, 0)),
            pl.BlockSpec((1, 16), lambda i: (0, 0)),
        ],
        out_specs=[
            pl.BlockSpec((NBLK, 16), lambda i: (i, 0)),
            pl.BlockSpec((NBLK, 16), lambda i: (i, 0)),
            pl.BlockSpec((NBLK, 16), lambda i: (i, 0)),
        ],
        out_shape=[
            jax.ShapeDtypeStruct((NPAD, 16), jnp.float32),
            jax.ShapeDtypeStruct((NPAD, 16), jnp.float32),
            jax.ShapeDtypeStruct((NPAD, 16), jnp.float32),
        ],
    )(xp, w1x, w1xz, w2xz, b1, b2)


def _gather_body(pre_hbm, src_hbm, out_hbm, idx_v, rows_v, sem):
    w = lax.axis_index("s") * 2 + lax.axis_index("c")
    base0 = w * EW

    def chunk(ci, _):
        base = base0 + ci * GCH
        pltpu.sync_copy(src_hbm.at[pl.ds(base, GCH)], idx_v)
        descs = []
        for j in range(GCH // GSUB):
            descs.append(pltpu.async_copy(
                pre_hbm.at[idx_v.at[pl.ds(j * GSUB, GSUB)]],
                rows_v.at[pl.ds(j * GSUB, GSUB), :],
                sem,
            ))
        for d in descs:
            d.wait()
        pltpu.sync_copy(rows_v, out_hbm.at[pl.ds(base, GCH)])
        return 0

    lax.fori_loop(0, EW // GCH, chunk, 0)


def _gather(pre, srcp):
    return pl.kernel(
        _gather_body,
        out_type=jax.ShapeDtypeStruct((EPAD, 16), jnp.float32),
        mesh=_mesh(),
        compiler_params=pltpu.CompilerParams(use_tc_tiling_on_sc=False),
        scratch_types=[
            pltpu.VMEM((GCH,), jnp.int32),
            pltpu.VMEM((GCH, 16), jnp.float32),
            pltpu.SemaphoreType.DMA,
        ],
    )(pre, srcp)


def _msg_body(g_ref, ea_ref, wea_ref, w1be_ref, w1bo_ref, b1be_ref,
              b1bo_ref, o_ref):
    eac = lax.dot_general(ea_ref[...], wea_ref[...], (((0,), (0,)), ((), ())))
    h1 = jnp.maximum(g_ref[...] + eac, 0.0)
    me = jnp.maximum(
        lax.dot_general(w1be_ref[...], h1, (((0,), (1,)), ((), ())))
        + b1be_ref[...], 0.0)
    mo = jnp.maximum(
        lax.dot_general(w1bo_ref[...], h1, (((0,), (1,)), ((), ())))
        + b1bo_ref[...], 0.0)
    ue = lax.shift_right_logical(
        lax.bitcast_convert_type(me.astype(jnp.bfloat16).astype(jnp.float32),
                                 jnp.int32), 16)
    uo = lax.shift_right_logical(
        lax.bitcast_convert_type(mo.astype(jnp.bfloat16).astype(jnp.float32),
                                 jnp.int32), 16)
    o_ref[...] = ue | lax.shift_left(uo, 16)


def _msg(gathered, eaT, wea, w1be, w1bo, b1be, b1bo):
    eblk = 4096
    return pl.pallas_call(
        _msg_body,
        grid=(EPAD // eblk,),
        in_specs=[
            pl.BlockSpec((eblk, 16), lambda i: (i, 0)),
            pl.BlockSpec((2, eblk), lambda i: (0, i)),
            pl.BlockSpec((2, 16), lambda i: (0, 0)),
            pl.BlockSpec((16, 16), lambda i: (0, 0)),
            pl.BlockSpec((16, 16), lambda i: (0, 0)),
            pl.BlockSpec((16, 1), lambda i: (0, 0)),
            pl.BlockSpec((16, 1), lambda i: (0, 0)),
        ],
        out_specs=pl.BlockSpec((16, eblk), lambda i: (0, i)),
        out_shape=jax.ShapeDtypeStruct((16, EPAD), jnp.int32),
    )(gathered, eaT, wea, w1be, w1bo, b1be, b1bo)


def _pmax(a, b):
    lo = jnp.maximum(a & 0xFFFF, b & 0xFFFF)
    hi = jnp.maximum(lax.shift_right_logical(a, 16),
                     lax.shift_right_logical(b, 16))
    return lo | lax.shift_left(hi, 16)


def _update_body(msg_hbm, dst_hbm, out_hbm,
                 table, dst0, dst1, msg0, msg1, sem0, sem1):
    s = lax.axis_index("s")
    c = lax.axis_index("c")
    cbase = c * (EPAD // UCH // 2)

    def start(ci, dstb, msgb, sem):
        eb = ci * UCH
        pltpu.async_copy(dst_hbm.at[pl.ds(eb, UCH)], dstb, sem)
        pltpu.async_copy(msg_hbm.at[s, pl.ds(eb, UCH)], msgb, sem)

    def drain(dstb, msgb, sem):
        pltpu.make_async_copy(dst_hbm.at[pl.ds(0, UCH)], dstb, sem).wait()
        pltpu.make_async_copy(msg_hbm.at[s, pl.ds(0, UCH)], msgb, sem).wait()

    def compute(dstb, msgb):
        def vreg(g, carry):
            acc, dstv, m, occ = carry
            dstv_n = dstb[pl.ds(g * 16 + 16, 16)]
            m_n = msgb[pl.ds(g * 16 + 16, 16)]
            occ_n, _last = plsc.scan_count(dstv_n)
            for r in (1, 2):
                msk = occ == r
                cur = plsc.load_gather(table, [dstv], mask=msk)
                plsc.store_scatter(table, [dstv], _pmax(cur, m), mask=msk)
            return (acc | (occ > 2).astype(jnp.int32), dstv_n, m_n, occ_n)

        dstv0 = dstb[pl.ds(0, 16)]
        m0 = msgb[pl.ds(0, 16)]
        occ0, _l0 = plsc.scan_count(dstv0)
        acc, dstv_l, m_l, occ_l = lax.fori_loop(
            0, UCH // 16 - 1, vreg,
            (jnp.zeros((16,), jnp.int32), dstv0, m0, occ0))
        for r in (1, 2):
            msk = occ_l == r
            cur = plsc.load_gather(table, [dstv_l], mask=msk)
            plsc.store_scatter(table, [dstv_l], _pmax(cur, m_l), mask=msk)
        acc = acc | (occ_l > 2).astype(jnp.int32)

        @pl.when(jnp.max(acc) > 0)
        def _():
            def vreg_slow(g, _):
                dstv = dstb[pl.ds(g * 16, 16)]
                m = msgb[pl.ds(g * 16, 16)]
                occ, _last = plsc.scan_count(dstv)
                mx = jnp.max(occ)

                def round_body(r):
                    msk = occ == r
                    cur = plsc.load_gather(table, [dstv], mask=msk)
                    plsc.store_scatter(table, [dstv], _pmax(cur, m),
                                       mask=msk)
                    return r + 1

                lax.while_loop(lambda r: r <= mx, round_body,
                               jnp.int32(3))
                return 0

            lax.fori_loop(0, UCH // 16, vreg_slow, 0)

    def init(i, _):
        table[pl.ds(i * 16, 16)] = jnp.zeros((16,), jnp.int32)
        return 0

    lax.fori_loop(0, NPAD // 16, init, 0)

    start(cbase, dst0, msg0, sem0)

    def pair(i, _):
        a = cbase + 2 * i
        start(a + 1, dst1, msg1, sem1)
        drain(dst0, msg0, sem0)
        compute(dst0, msg0)

        @pl.when(i < UPAIRS_H - 1)
        def _():
            start(a + 2, dst0, msg0, sem0)

        drain(dst1, msg1, sem1)
        compute(dst1, msg1)
        return 0

    lax.fori_loop(0, UPAIRS_H, pair, 0)
    pltpu.sync_copy(table, out_hbm.at[c, s])


def _update(msgsP, dstp):
    return pl.kernel(
        _update_body,
        out_type=jax.ShapeDtypeStruct((2, 16, NPAD), jnp.int32),
        mesh=_mesh(),
        compiler_params=pltpu.CompilerParams(needs_layout_passes=False),
        scratch_types=[
            pltpu.VMEM((NPAD,), jnp.int32),
            pltpu.VMEM((UCH,), jnp.int32),
            pltpu.VMEM((UCH,), jnp.int32),
            pltpu.VMEM((UCH,), jnp.int32),
            pltpu.VMEM((UCH,), jnp.int32),
            pltpu.SemaphoreType.DMA,
            pltpu.SemaphoreType.DMA,
        ],
    )(msgsP, dstp)


def _agg_terms(aggP_ref, w2e_ref, w2og_ref):
    a0 = aggP_ref[0]
    a1 = aggP_ref[1]
    ev = jnp.maximum(
        jnp.maximum(lax.bitcast_convert_type(lax.shift_left(a0, 16),
                                             jnp.float32),
                    lax.bitcast_convert_type(lax.shift_left(a1, 16),
                                             jnp.float32)), 0.0)
    od = jnp.maximum(
        jnp.maximum(
            lax.bitcast_convert_type(a0 & -65536, jnp.float32),
            lax.bitcast_convert_type(a1 & -65536, jnp.float32)), 0.0)
    return (lax.dot_general(ev, w2e_ref[...], (((0,), (0,)), ((), ())))
            + lax.dot_general(od, w2og_ref[...], (((0,), (0,)), ((), ()))))


def _mlp2_body(aggP_ref, w2b_ref, w1b_ref, x11_ref, w2e_ref, w2og_ref,
               w2o_ref, b2o_ref, w1a11_ref, w2a11_ref, pre_ref, comb_ref):
    t = jnp.maximum(
        w2b_ref[...] + x11_ref[...] * w2a11_ref[...]
        + _agg_terms(aggP_ref, w2e_ref, w2og_ref), 0.0)
    comb = jnp.maximum(t @ w2o_ref[...] + b2o_ref[...], 0.0)
    pre_ref[...] = w1b_ref[...] + comb * w1a11_ref[...]
    comb_ref[...] = comb


def _mlp2(aggP, w2base, w1base, x11, w2e, w2og, w2o, b2o, w1a11, w2a11):
    return pl.pallas_call(
        _mlp2_body,
        grid=(NPAD // NBLK,),
        in_specs=[
            pl.BlockSpec((2, 16, NBLK), lambda i: (0, 0, i)),
            pl.BlockSpec((NBLK, 16), lambda i: (i, 0)),
            pl.BlockSpec((NBLK, 16), lambda i: (i, 0)),
            pl.BlockSpec((NBLK, 1), lambda i: (i, 0)),
            pl.BlockSpec((16, 16), lambda i: (0, 0)),
            pl.BlockSpec((16, 16), lambda i: (0, 0)),
            pl.BlockSpec((16, 1), lambda i: (0, 0)),
            pl.BlockSpec((1, 1), lambda i: (0, 0)),
            pl.BlockSpec((1, 16), lambda i: (0, 0)),
            pl.BlockSpec((1, 16), lambda i: (0, 0)),
        ],
        out_specs=[
            pl.BlockSpec((NBLK, 16), lambda i: (i, 0)),
            pl.BlockSpec((NBLK, 1), lambda i: (i, 0)),
        ],
        out_shape=[
            jax.ShapeDtypeStruct((NPAD, 16), jnp.float32),
            jax.ShapeDtypeStruct((NPAD, 1), jnp.float32),
        ],
    )(aggP, w2base, w1base, x11, w2e, w2og, w2o, b2o, w1a11, w2a11)


def _final_body(aggP_ref, w2b_ref, x_ref, x11_ref, w2e_ref, w2og_ref,
                w2o_ref, b2o_ref, w2a11_ref, o_ref):
    t = jnp.maximum(
        w2b_ref[...] + x11_ref[...] * w2a11_ref[...]
        + _agg_terms(aggP_ref, w2e_ref, w2og_ref), 0.0)
    comb = jnp.maximum(t @ w2o_ref[...] + b2o_ref[...], 0.0)
    o_ref[...] = jnp.concatenate([x_ref[...][:, :11], comb], axis=1)


def _final(aggP, w2base, xp, x11, w2e, w2og, w2o, b2o, w2a11):
    return pl.pallas_call(
        _final_body,
        grid=(NPAD // NBLK,),
        in_specs=[
            pl.BlockSpec((2, 16, NBLK), lambda i: (0, 0, i)),
            pl.BlockSpec((NBLK, 16), lambda i: (i, 0)),
            pl.BlockSpec((NBLK, 12), lambda i: (i, 0)),
            pl.BlockSpec((NBLK, 1), lambda i: (i, 0)),
            pl.BlockSpec((16, 16), lambda i: (0, 0)),
            pl.BlockSpec((16, 16), lambda i: (0, 0)),
            pl.BlockSpec((16, 1), lambda i: (0, 0)),
            pl.BlockSpec((1, 1), lambda i: (0, 0)),
            pl.BlockSpec((1, 16), lambda i: (0, 0)),
        ],
        out_specs=pl.BlockSpec((NBLK, 12), lambda i: (i, 0)),
        out_shape=jax.ShapeDtypeStruct((NPAD, 12), jnp.float32),
    )(aggP, w2base, xp, x11, w2e, w2og, w2o, b2o, w2a11)


def kernel(x, edge_index, edge_attr, W1a, b1a, W1b, b1b, W2a, b2a, W2b, b2b):
    src = edge_index[0].astype(jnp.int32)
    dst = edge_index[1].astype(jnp.int32)

    xp = jnp.pad(x, ((0, NPAD - NN), (0, 0)))
    srcp = jnp.pad(src, (0, EPAD - NE))
    dstp = jnp.pad(dst, (0, EPAD - NE), constant_values=TRASH)
    eaT = jnp.pad(edge_attr.T, ((0, 0), (0, EPAD - NE)))

    zrow = jnp.zeros((1, 16), jnp.float32)
    w1x = W1a[:12]
    w1xz = jnp.concatenate([W1a[:11], zrow], axis=0)
    wea = W1a[12:14]
    w2xz = jnp.concatenate([W2a[:11], zrow], axis=0)
    w2agg = W2a[12:44]
    b1 = b1a[None, :]
    b2 = b2a[None, :]
    b2o = b2b[None, :]
    w1a11 = W1a[11:12]
    w2a11 = W2a[11:12]
    w1be = W1b[:, 0::2]
    w1bo = W1b[:, 1::2]
    b1be = b1b[0::2][:, None]
    b1bo = b1b[1::2][:, None]
    w2e = w2agg[0::2]
    w2og = w2agg[1::2]

    pre, w1base, w2base = _node_premix(xp, w1x, w1xz, w2xz, b1, b2)
    x11 = xp[:, 11:12]

    for layer in range(3):
        gathered = _gather(pre, srcp)
        msgsP = _msg(gathered, eaT, wea, w1be, w1bo, b1be, b1bo)
        aggP = _update(msgsP, dstp)
        if layer < 2:
            pre, x11 = _mlp2(aggP, w2base, w1base, x11, w2e, w2og, W2b,
                             b2o, w1a11, w2a11)
        else:
            outp = _final(aggP, w2base, xp, x11, w2e, w2og, W2b, b2o,
                          w2a11)

    return outp[:NN]


# speedup vs baseline: 5.2829x; 1.0151x over previous
import functools

import jax
import jax.numpy as jnp
from jax import lax
from jax.experimental import pallas as pl
from jax.experimental.pallas import tpu as pltpu
from jax.experimental.pallas import tpu_sc as plsc

NN = 100000
NE = 6400000
NBLK = 2048
NPAD = 100352
EPAD = 6422528
NWORK = 32
EW = EPAD // NWORK
GCH = 1024
GSUB = 128
UCH = 4096
UPAIRS_H = EPAD // (4 * UCH)
TRASH = NN

_mesh = functools.partial(
    plsc.VectorSubcoreMesh,
    core_axis_name="c",
    subcore_axis_name="s",
    num_cores=2,
    num_subcores=16,
)


def _node_body(x_ref, w1x_ref, w1xz_ref, w2xz_ref, b1_ref, b2_ref,
               pre0_ref, w1b_ref, w2b_ref):
    xb = x_ref[...]
    pre0_ref[...] = (xb @ w1x_ref[...] + b1_ref[...]).astype(jnp.bfloat16)
    w1b_ref[...] = xb @ w1xz_ref[...] + b1_ref[...]
    w2b_ref[...] = xb @ w2xz_ref[...] + b2_ref[...]


def _node_premix(xp, w1x, w1xz, w2xz, b1, b2):
    return pl.pallas_call(
        _node_body,
        grid=(NPAD // NBLK,),
        in_specs=[
            pl.BlockSpec((NBLK, 12), lambda i: (i, 0)),
            pl.BlockSpec((12, 16), lambda i: (0, 0)),
            pl.BlockSpec((12, 16), lambda i: (0, 0)),
            pl.BlockSpec((12, 16), lambda i: (0, 0)),
            pl.BlockSpec((1, 16), lambda i: (0, 0)),
            pl.BlockSpec((1, 16), lambda i: (0, 0)),
        ],
        out_specs=[
            pl.BlockSpec((NBLK, 16), lambda i: (i, 0)),
            pl.BlockSpec((NBLK, 16), lambda i: (i, 0)),
            pl.BlockSpec((NBLK, 16), lambda i: (i, 0)),
        ],
        out_shape=[
            jax.ShapeDtypeStruct((NPAD, 16), jnp.bfloat16),
            jax.ShapeDtypeStruct((NPAD, 16), jnp.float32),
            jax.ShapeDtypeStruct((NPAD, 16), jnp.float32),
        ],
    )(xp, w1x, w1xz, w2xz, b1, b2)


def _gather_body(pre_hbm, src_hbm, out_hbm, idx_v, rows_v, sem):
    w = lax.axis_index("s") * 2 + lax.axis_index("c")
    base0 = w * EW

    def chunk(ci, _):
        base = base0 + ci * GCH
        pltpu.sync_copy(src_hbm.at[pl.ds(base, GCH)], idx_v)
        descs = []
        for j in range(GCH // GSUB):
            descs.append(pltpu.async_copy(
                pre_hbm.at[idx_v.at[pl.ds(j * GSUB, GSUB)]],
                rows_v.at[pl.ds(j * GSUB, GSUB), :],
                sem,
            ))
        for d in descs:
            d.wait()
        pltpu.sync_copy(rows_v, out_hbm.at[pl.ds(base, GCH)])
        return 0

    lax.fori_loop(0, EW // GCH, chunk, 0)


def _gather(pre, srcp):
    return pl.kernel(
        _gather_body,
        out_type=jax.ShapeDtypeStruct((EPAD, 16), jnp.bfloat16),
        mesh=_mesh(),
        compiler_params=pltpu.CompilerParams(use_tc_tiling_on_sc=False),
        scratch_types=[
            pltpu.VMEM((GCH,), jnp.int32),
            pltpu.VMEM((GCH, 16), jnp.bfloat16),
            pltpu.SemaphoreType.DMA,
        ],
    )(pre, srcp)


def _msg_body(g_ref, ea_ref, wea_ref, w1be_ref, w1bo_ref, b1be_ref,
              b1bo_ref, o_ref):
    eac = lax.dot_general(ea_ref[...], wea_ref[...], (((0,), (0,)), ((), ())))
    h1 = jnp.maximum(g_ref[...].astype(jnp.float32) + eac, 0.0)
    me = jnp.maximum(
        lax.dot_general(w1be_ref[...], h1, (((0,), (1,)), ((), ())))
        + b1be_ref[...], 0.0)
    mo = jnp.maximum(
        lax.dot_general(w1bo_ref[...], h1, (((0,), (1,)), ((), ())))
        + b1bo_ref[...], 0.0)
    ue = lax.shift_right_logical(
        lax.bitcast_convert_type(me.astype(jnp.bfloat16).astype(jnp.float32),
                                 jnp.int32), 16)
    uo = lax.shift_right_logical(
        lax.bitcast_convert_type(mo.astype(jnp.bfloat16).astype(jnp.float32),
                                 jnp.int32), 16)
    o_ref[...] = ue | lax.shift_left(uo, 16)


def _msg(gathered, eaT, wea, w1be, w1bo, b1be, b1bo):
    eblk = 4096
    return pl.pallas_call(
        _msg_body,
        grid=(EPAD // eblk,),
        in_specs=[
            pl.BlockSpec((eblk, 16), lambda i: (i, 0)),
            pl.BlockSpec((2, eblk), lambda i: (0, i)),
            pl.BlockSpec((2, 16), lambda i: (0, 0)),
            pl.BlockSpec((16, 16), lambda i: (0, 0)),
            pl.BlockSpec((16, 16), lambda i: (0, 0)),
            pl.BlockSpec((16, 1), lambda i: (0, 0)),
            pl.BlockSpec((16, 1), lambda i: (0, 0)),
        ],
        out_specs=pl.BlockSpec((16, eblk), lambda i: (0, i)),
        out_shape=jax.ShapeDtypeStruct((16, EPAD), jnp.int32),
    )(gathered, eaT, wea, w1be, w1bo, b1be, b1bo)


def _pmax(a, b):
    lo = jnp.maximum(a & 0xFFFF, b & 0xFFFF)
    hi = jnp.maximum(lax.shift_right_logical(a, 16),
                     lax.shift_right_logical(b, 16))
    return lo | lax.shift_left(hi, 16)


def _update_body(msg_hbm, dst_hbm, out_hbm,
                 table, dst0, dst1, msg0, msg1, sem0, sem1):
    s = lax.axis_index("s")
    c = lax.axis_index("c")
    cbase = c * (EPAD // UCH // 2)

    def start(ci, dstb, msgb, sem):
        eb = ci * UCH
        pltpu.async_copy(dst_hbm.at[pl.ds(eb, UCH)], dstb, sem)
        pltpu.async_copy(msg_hbm.at[s, pl.ds(eb, UCH)], msgb, sem)

    def drain(dstb, msgb, sem):
        pltpu.make_async_copy(dst_hbm.at[pl.ds(0, UCH)], dstb, sem).wait()
        pltpu.make_async_copy(msg_hbm.at[s, pl.ds(0, UCH)], msgb, sem).wait()

    def compute(dstb, msgb):
        def vreg(g, carry):
            acc, dstv, m, occ = carry
            dstv_n = dstb[pl.ds(g * 16 + 16, 16)]
            m_n = msgb[pl.ds(g * 16 + 16, 16)]
            occ_n, _last = plsc.scan_count(dstv_n)
            for r in (1, 2):
                msk = occ == r
                cur = plsc.load_gather(table, [dstv], mask=msk)
                plsc.store_scatter(table, [dstv], _pmax(cur, m), mask=msk)
            return (acc | (occ > 2).astype(jnp.int32), dstv_n, m_n, occ_n)

        dstv0 = dstb[pl.ds(0, 16)]
        m0 = msgb[pl.ds(0, 16)]
        occ0, _l0 = plsc.scan_count(dstv0)
        acc, dstv_l, m_l, occ_l = lax.fori_loop(
            0, UCH // 16 - 1, vreg,
            (jnp.zeros((16,), jnp.int32), dstv0, m0, occ0))
        for r in (1, 2):
            msk = occ_l == r
            cur = plsc.load_gather(table, [dstv_l], mask=msk)
            plsc.store_scatter(table, [dstv_l], _pmax(cur, m_l), mask=msk)
        acc = acc | (occ_l > 2).astype(jnp.int32)

        @pl.when(jnp.max(acc) > 0)
        def _():
            def vreg_slow(g, _):
                dstv = dstb[pl.ds(g * 16, 16)]
                m = msgb[pl.ds(g * 16, 16)]
                occ, _last = plsc.scan_count(dstv)
                mx = jnp.max(occ)

                def round_body(r):
                    msk = occ == r
                    cur = plsc.load_gather(table, [dstv], mask=msk)
                    plsc.store_scatter(table, [dstv], _pmax(cur, m),
                                       mask=msk)
                    return r + 1

                lax.while_loop(lambda r: r <= mx, round_body,
                               jnp.int32(3))
                return 0

            lax.fori_loop(0, UCH // 16, vreg_slow, 0)

    def init(i, _):
        table[pl.ds(i * 16, 16)] = jnp.zeros((16,), jnp.int32)
        return 0

    lax.fori_loop(0, NPAD // 16, init, 0)

    start(cbase, dst0, msg0, sem0)

    def pair(i, _):
        a = cbase + 2 * i
        start(a + 1, dst1, msg1, sem1)
        drain(dst0, msg0, sem0)
        compute(dst0, msg0)

        @pl.when(i < UPAIRS_H - 1)
        def _():
            start(a + 2, dst0, msg0, sem0)

        drain(dst1, msg1, sem1)
        compute(dst1, msg1)
        return 0

    lax.fori_loop(0, UPAIRS_H, pair, 0)
    pltpu.sync_copy(table, out_hbm.at[c, s])


def _update(msgsP, dstp):
    return pl.kernel(
        _update_body,
        out_type=jax.ShapeDtypeStruct((2, 16, NPAD), jnp.int32),
        mesh=_mesh(),
        compiler_params=pltpu.CompilerParams(needs_layout_passes=False),
        scratch_types=[
            pltpu.VMEM((NPAD,), jnp.int32),
            pltpu.VMEM((UCH,), jnp.int32),
            pltpu.VMEM((UCH,), jnp.int32),
            pltpu.VMEM((UCH,), jnp.int32),
            pltpu.VMEM((UCH,), jnp.int32),
            pltpu.SemaphoreType.DMA,
            pltpu.SemaphoreType.DMA,
        ],
    )(msgsP, dstp)


def _agg_terms(aggP_ref, w2e_ref, w2og_ref):
    a0 = aggP_ref[0]
    a1 = aggP_ref[1]
    ev = jnp.maximum(
        jnp.maximum(lax.bitcast_convert_type(lax.shift_left(a0, 16),
                                             jnp.float32),
                    lax.bitcast_convert_type(lax.shift_left(a1, 16),
                                             jnp.float32)), 0.0)
    od = jnp.maximum(
        jnp.maximum(
            lax.bitcast_convert_type(a0 & -65536, jnp.float32),
            lax.bitcast_convert_type(a1 & -65536, jnp.float32)), 0.0)
    return (lax.dot_general(ev, w2e_ref[...], (((0,), (0,)), ((), ())))
            + lax.dot_general(od, w2og_ref[...], (((0,), (0,)), ((), ()))))


def _mlp2_body(aggP_ref, w2b_ref, w1b_ref, x11_ref, w2e_ref, w2og_ref,
               w2o_ref, b2o_ref, w1a11_ref, w2a11_ref, pre_ref, comb_ref):
    t = jnp.maximum(
        w2b_ref[...] + x11_ref[...] * w2a11_ref[...]
        + _agg_terms(aggP_ref, w2e_ref, w2og_ref), 0.0)
    comb = jnp.maximum(t @ w2o_ref[...] + b2o_ref[...], 0.0)
    pre_ref[...] = (w1b_ref[...] + comb * w1a11_ref[...]).astype(jnp.bfloat16)
    comb_ref[...] = comb


def _mlp2(aggP, w2base, w1base, x11, w2e, w2og, w2o, b2o, w1a11, w2a11):
    return pl.pallas_call(
        _mlp2_body,
        grid=(NPAD // NBLK,),
        in_specs=[
            pl.BlockSpec((2, 16, NBLK), lambda i: (0, 0, i)),
            pl.BlockSpec((NBLK, 16), lambda i: (i, 0)),
            pl.BlockSpec((NBLK, 16), lambda i: (i, 0)),
            pl.BlockSpec((NBLK, 1), lambda i: (i, 0)),
            pl.BlockSpec((16, 16), lambda i: (0, 0)),
            pl.BlockSpec((16, 16), lambda i: (0, 0)),
            pl.BlockSpec((16, 1), lambda i: (0, 0)),
            pl.BlockSpec((1, 1), lambda i: (0, 0)),
            pl.BlockSpec((1, 16), lambda i: (0, 0)),
            pl.BlockSpec((1, 16), lambda i: (0, 0)),
        ],
        out_specs=[
            pl.BlockSpec((NBLK, 16), lambda i: (i, 0)),
            pl.BlockSpec((NBLK, 1), lambda i: (i, 0)),
        ],
        out_shape=[
            jax.ShapeDtypeStruct((NPAD, 16), jnp.bfloat16),
            jax.ShapeDtypeStruct((NPAD, 1), jnp.float32),
        ],
    )(aggP, w2base, w1base, x11, w2e, w2og, w2o, b2o, w1a11, w2a11)


def _final_body(aggP_ref, w2b_ref, x_ref, x11_ref, w2e_ref, w2og_ref,
                w2o_ref, b2o_ref, w2a11_ref, o_ref):
    t = jnp.maximum(
        w2b_ref[...] + x11_ref[...] * w2a11_ref[...]
        + _agg_terms(aggP_ref, w2e_ref, w2og_ref), 0.0)
    comb = jnp.maximum(t @ w2o_ref[...] + b2o_ref[...], 0.0)
    o_ref[...] = jnp.concatenate([x_ref[...][:, :11], comb], axis=1)


def _final(aggP, w2base, xp, x11, w2e, w2og, w2o, b2o, w2a11):
    return pl.pallas_call(
        _final_body,
        grid=(NPAD // NBLK,),
        in_specs=[
            pl.BlockSpec((2, 16, NBLK), lambda i: (0, 0, i)),
            pl.BlockSpec((NBLK, 16), lambda i: (i, 0)),
            pl.BlockSpec((NBLK, 12), lambda i: (i, 0)),
            pl.BlockSpec((NBLK, 1), lambda i: (i, 0)),
            pl.BlockSpec((16, 16), lambda i: (0, 0)),
            pl.BlockSpec((16, 16), lambda i: (0, 0)),
            pl.BlockSpec((16, 1), lambda i: (0, 0)),
            pl.BlockSpec((1, 1), lambda i: (0, 0)),
            pl.BlockSpec((1, 16), lambda i: (0, 0)),
        ],
        out_specs=pl.BlockSpec((NBLK, 12), lambda i: (i, 0)),
        out_shape=jax.ShapeDtypeStruct((NPAD, 12), jnp.float32),
    )(aggP, w2base, xp, x11, w2e, w2og, w2o, b2o, w2a11)


def kernel(x, edge_index, edge_attr, W1a, b1a, W1b, b1b, W2a, b2a, W2b, b2b):
    src = edge_index[0].astype(jnp.int32)
    dst = edge_index[1].astype(jnp.int32)

    xp = jnp.pad(x, ((0, NPAD - NN), (0, 0)))
    srcp = jnp.pad(src, (0, EPAD - NE))
    dstp = jnp.pad(dst, (0, EPAD - NE), constant_values=TRASH)
    eaT = jnp.pad(edge_attr.T, ((0, 0), (0, EPAD - NE)))

    zrow = jnp.zeros((1, 16), jnp.float32)
    w1x = W1a[:12]
    w1xz = jnp.concatenate([W1a[:11], zrow], axis=0)
    wea = W1a[12:14]
    w2xz = jnp.concatenate([W2a[:11], zrow], axis=0)
    w2agg = W2a[12:44]
    b1 = b1a[None, :]
    b2 = b2a[None, :]
    b2o = b2b[None, :]
    w1a11 = W1a[11:12]
    w2a11 = W2a[11:12]
    w1be = W1b[:, 0::2]
    w1bo = W1b[:, 1::2]
    b1be = b1b[0::2][:, None]
    b1bo = b1b[1::2][:, None]
    w2e = w2agg[0::2]
    w2og = w2agg[1::2]

    pre, w1base, w2base = _node_premix(xp, w1x, w1xz, w2xz, b1, b2)
    x11 = xp[:, 11:12]

    for layer in range(3):
        gathered = _gather(pre, srcp)
        msgsP = _msg(gathered, eaT, wea, w1be, w1bo, b1be, b1bo)
        aggP = _update(msgsP, dstp)
        if layer < 2:
            pre, x11 = _mlp2(aggP, w2base, w1base, x11, w2e, w2og, W2b,
                             b2o, w1a11, w2a11)
        else:
            outp = _final(aggP, w2base, xp, x11, w2e, w2og, W2b, b2o,
                          w2a11)

    return outp[:NN]
